# baseline (speedup 1.0000x reference)
import jax
import jax.numpy as jnp
from jax.experimental import pallas as pl
from jax.experimental.pallas import tpu as pltpu

N = 50000
B = 64
H = 128
OUT = 256
NL = 5
FL = 3
D = (H // 2) * FL

_BN_S = 1.0 / jnp.sqrt(1.0 + 1e-5)


def _bn(h, g, b):
    return g * h * _BN_S + b


def _head_kernel(pooled_ref, w1_ref, b1_ref, w2_ref, b2_ref, o_ref):
    o = jax.nn.relu(jnp.dot(pooled_ref[...], w1_ref[...],
                            preferred_element_type=jnp.float32) + b1_ref[...])
    o = jax.nn.relu(jnp.dot(o, w2_ref[...],
                            preferred_element_type=jnp.float32) + b2_ref[...])
    o_ref[...] = o


def kernel(x, edge_index, positions, batch, params):
    p = params
    pe = jax.nn.relu(positions @ p['pos_W1'] + p['pos_b1'])
    pe = jax.nn.relu(pe @ p['pos_W2'] + p['pos_b2'])
    h = jnp.concatenate([x, pe], axis=1)
    h = jax.nn.relu(_bn(h @ p['proj_W'] + p['proj_b'], p['proj_g'], p['proj_bb']))
    h_res = h @ p['res_W'] + p['res_b']
    src = edge_index[0]
    dst = edge_index[1]
    n = h.shape[0]
    layer_feats = []
    for i in range(NL):
        agg = jax.ops.segment_sum(h[src], dst, num_segments=n)
        t = h + agg
        t = jax.nn.relu(t @ p['gin_W1'][i] + p['gin_b1'][i])
        t = jax.nn.relu(t @ p['gin_W2'][i] + p['gin_b2'][i])
        t = _bn(t, p['gin_g'][i], p['gin_bb'][i])
        if i == 0:
            t = t + h_res
        t = _bn(t, p['bn_g'][i], p['bn_b'][i])
        h = jax.nn.relu(t)
        if i >= NL - FL:
            layer_feats.append(h)
    fused = [jax.nn.relu(f @ p['fus_W'][j] + p['fus_b'][j]) for j, f in enumerate(layer_feats)]
    xf = jnp.concatenate(fused, axis=1)
    a = jnp.tanh(xf @ p['attn_W1'] + p['attn_b1'])
    a = a @ p['attn_W2'] + p['attn_b2']
    aw = jax.nn.softmax(a, axis=0)
    counts = jnp.maximum(jax.ops.segment_sum(jnp.ones((n,), jnp.float32), batch, num_segments=B), 1.0)[:, None]
    wsum = jax.ops.segment_sum(xf * aw, batch, num_segments=B) / counts
    gmean = jax.ops.segment_sum(xf, batch, num_segments=B) / counts
    gmax = jax.ops.segment_max(xf, batch, num_segments=B)
    pooled = jnp.concatenate([wsum, gmean, gmax], axis=1)
    o = pl.pallas_call(
        _head_kernel,
        out_shape=jax.ShapeDtypeStruct((B, OUT), jnp.float32),
    )(pooled, p['out_W1'], p['out_b1'][None, :], p['out_W2'], p['out_b2'][None, :])
    return o


# speedup vs baseline: 3.0891x; 3.0891x over previous
import functools

import jax
import jax.numpy as jnp
from jax import lax
from jax.experimental import pallas as pl
from jax.experimental.pallas import tpu as pltpu
from jax.experimental.pallas import tpu_sc as plsc

N = 50000
E = 800000
B = 64
H = 128
OUT = 256
NL = 5
FL = 3
D = (H // 2) * FL

_BN_S = float((1.0 + 1e-5) ** -0.5)

NSUB = 16
NCORE = 2
CHUNK_W = 32
NCHUNK = H // CHUNK_W
BATCH = 125
KSUB = 4
G = KSUB * BATCH
EP = E // NSUB
NSTEP = EP // G
STRIPE = 3200
STRIPE_LAST = N - STRIPE * (NSUB - 1)


def _bn(h, g, b):
    return g * h * _BN_S + b


def _agg_body(h4, sidx, didx, zrows, out, acc, rows, sv, dv, gsem):
    cid = lax.axis_index("c")
    sid = lax.axis_index("s")
    last = NSUB - 1
    for step in range(NCHUNK // NCORE):
        c = cid + NCORE * step
        @pl.when(sid < last)
        def _():
            pltpu.sync_copy(zrows, acc.at[pl.ds(sid * STRIPE, STRIPE)])

        @pl.when(sid == last)
        def _():
            pltpu.sync_copy(zrows.at[pl.ds(0, STRIPE_LAST)],
                            acc.at[pl.ds(sid * STRIPE, STRIPE_LAST)])

        plsc.subcore_barrier()

        def body(i, carry):
            row_s = c * (E // BATCH) + sid * (EP // BATCH) + i * KSUB
            row_d = sid * (EP // BATCH) + i * KSUB
            pltpu.sync_copy(sidx.at[pl.ds(row_s, KSUB)], sv)
            pltpu.sync_copy(didx.at[pl.ds(row_d, KSUB)], dv)
            cps = [
                pltpu.async_copy(h4.at[sv.at[j]],
                                 rows.at[pl.ds(j * BATCH, BATCH)], gsem)
                for j in range(KSUB)
            ]
            for cp in cps:
                cp.wait()
            for j in range(KSUB):
                pltpu.sync_copy(rows.at[pl.ds(j * BATCH, BATCH)],
                                acc.at[dv.at[j]], add=True)
            return carry

        lax.fori_loop(0, NSTEP, body, None)
        plsc.subcore_barrier()

        @pl.when(sid < last)
        def _():
            pltpu.sync_copy(acc.at[pl.ds(sid * STRIPE, STRIPE)],
                            out.at[pl.ds(c * N + sid * STRIPE, STRIPE)])

        @pl.when(sid == last)
        def _():
            pltpu.sync_copy(acc.at[pl.ds(sid * STRIPE, STRIPE_LAST)],
                            out.at[pl.ds(c * N + sid * STRIPE, STRIPE_LAST)])

        plsc.subcore_barrier()


@functools.partial(jax.jit, static_argnames=())
def _sc_segment_sum(h4, sidx, didx, zrows):
    mesh = plsc.VectorSubcoreMesh(core_axis_name="c", subcore_axis_name="s")
    fn = pl.kernel(
        _agg_body,
        out_type=jax.ShapeDtypeStruct((NCHUNK * N, CHUNK_W), jnp.float32),
        mesh=mesh,
        scratch_types=[
            pltpu.VMEM_SHARED((N, CHUNK_W), jnp.float32),
            pltpu.VMEM((G, CHUNK_W), jnp.float32),
            pltpu.VMEM((KSUB, BATCH), jnp.int32),
            pltpu.VMEM((KSUB, BATCH), jnp.int32),
            pltpu.SemaphoreType.DMA,
        ],
        compiler_params=pltpu.CompilerParams(use_tc_tiling_on_sc=False),
    )
    return fn(h4, sidx, didx, zrows)


def _chunked(h):
    return h.reshape(N, NCHUNK, CHUNK_W).transpose(1, 0, 2).reshape(NCHUNK * N, CHUNK_W)


def _unchunked(h4):
    return h4.reshape(NCHUNK, N, CHUNK_W).transpose(1, 0, 2).reshape(N, H)


def kernel(x, edge_index, positions, batch, params):
    p = params
    src = edge_index[0].astype(jnp.int32)
    dst = edge_index[1].astype(jnp.int32)
    offs = (jnp.arange(NCHUNK, dtype=jnp.int32) * N)[:, None]
    sidx = (src[None, :] + offs).reshape(NCHUNK * E // BATCH, BATCH)
    didx = dst.reshape(E // BATCH, BATCH)
    zrows = jnp.zeros((STRIPE, CHUNK_W), jnp.float32)

    pe = jax.nn.relu(positions @ p['pos_W1'] + p['pos_b1'])
    pe = jax.nn.relu(pe @ p['pos_W2'] + p['pos_b2'])
    h = jnp.concatenate([x, pe], axis=1)
    h = jax.nn.relu(_bn(h @ p['proj_W'] + p['proj_b'], p['proj_g'], p['proj_bb']))
    h_res = h @ p['res_W'] + p['res_b']
    layer_feats = []
    for i in range(NL):
        agg = _unchunked(_sc_segment_sum(_chunked(h), sidx, didx, zrows))
        t = h + agg
        t = jax.nn.relu(t @ p['gin_W1'][i] + p['gin_b1'][i])
        t = jax.nn.relu(t @ p['gin_W2'][i] + p['gin_b2'][i])
        t = _bn(t, p['gin_g'][i], p['gin_bb'][i])
        if i == 0:
            t = t + h_res
        t = _bn(t, p['bn_g'][i], p['bn_b'][i])
        h = jax.nn.relu(t)
        if i >= NL - FL:
            layer_feats.append(h)
    fused = [jax.nn.relu(f @ p['fus_W'][j] + p['fus_b'][j]) for j, f in enumerate(layer_feats)]
    xf = jnp.concatenate(fused, axis=1)
    a = jnp.tanh(xf @ p['attn_W1'] + p['attn_b1'])
    a = a @ p['attn_W2'] + p['attn_b2']
    aw = jax.nn.softmax(a, axis=0)
    counts = jnp.maximum(jax.ops.segment_sum(jnp.ones((N,), jnp.float32), batch, num_segments=B), 1.0)[:, None]
    wsum = jax.ops.segment_sum(xf * aw, batch, num_segments=B) / counts
    gmean = jax.ops.segment_sum(xf, batch, num_segments=B) / counts
    gmax = jax.ops.segment_max(xf, batch, num_segments=B)
    pooled = jnp.concatenate([wsum, gmean, gmax], axis=1)
    o = jax.nn.relu(pooled @ p['out_W1'] + p['out_b1'])
    o = jax.nn.relu(o @ p['out_W2'] + p['out_b2'])
    return o


# speedup vs baseline: 3.6349x; 1.1767x over previous
import functools

import jax
import jax.numpy as jnp
from jax import lax
from jax.experimental import pallas as pl
from jax.experimental.pallas import tpu as pltpu
from jax.experimental.pallas import tpu_sc as plsc

N = 50000
E = 800000
B = 64
H = 128
OUT = 256
NL = 5
FL = 3
D = (H // 2) * FL

_BN_S = float((1.0 + 1e-5) ** -0.5)
NEG_BIG = float(jnp.finfo(jnp.float32).min)

NSUB = 16
NCORE = 2
CHUNK_W = 32
NCHUNK = H // CHUNK_W
BATCH = 125
KSUB = 4
G = KSUB * BATCH
EP = E // NSUB
NSTEP = EP // G
STRIPE = 3200
STRIPE_LAST = N - STRIPE * (NSUB - 1)


def _agg_body(h4, sidx, didx, zrows, out, acc, rows, sv, dv, gsem):
    cid = lax.axis_index("c")
    sid = lax.axis_index("s")
    last = NSUB - 1
    for step in range(NCHUNK // NCORE):
        c = cid + NCORE * step
        @pl.when(sid < last)
        def _():
            pltpu.sync_copy(zrows, acc.at[pl.ds(sid * STRIPE, STRIPE)])

        @pl.when(sid == last)
        def _():
            pltpu.sync_copy(zrows.at[pl.ds(0, STRIPE_LAST)],
                            acc.at[pl.ds(sid * STRIPE, STRIPE_LAST)])

        plsc.subcore_barrier()

        def body(i, carry):
            row_s = c * (E // BATCH) + sid * (EP // BATCH) + i * KSUB
            row_d = sid * (EP // BATCH) + i * KSUB
            pltpu.sync_copy(sidx.at[pl.ds(row_s, KSUB)], sv)
            pltpu.sync_copy(didx.at[pl.ds(row_d, KSUB)], dv)
            cps = [
                pltpu.async_copy(h4.at[sv.at[j]],
                                 rows.at[pl.ds(j * BATCH, BATCH)], gsem)
                for j in range(KSUB)
            ]
            for cp in cps:
                cp.wait()
            for j in range(KSUB):
                pltpu.sync_copy(rows.at[pl.ds(j * BATCH, BATCH)],
                                acc.at[dv.at[j]], add=True)
            return carry

        lax.fori_loop(0, NSTEP, body, None)
        plsc.subcore_barrier()

        @pl.when(sid < last)
        def _():
            pltpu.sync_copy(acc.at[pl.ds(sid * STRIPE, STRIPE)],
                            out.at[pl.ds(c * N + sid * STRIPE, STRIPE)])

        @pl.when(sid == last)
        def _():
            pltpu.sync_copy(acc.at[pl.ds(sid * STRIPE, STRIPE_LAST)],
                            out.at[pl.ds(c * N + sid * STRIPE, STRIPE_LAST)])

        plsc.subcore_barrier()


def _sc_segment_sum(h4, sidx, didx, zrows):
    mesh = plsc.VectorSubcoreMesh(core_axis_name="c", subcore_axis_name="s")
    fn = pl.kernel(
        _agg_body,
        out_type=jax.ShapeDtypeStruct((NCHUNK * N, CHUNK_W), jnp.float32),
        mesh=mesh,
        scratch_types=[
            pltpu.VMEM_SHARED((N, CHUNK_W), jnp.float32),
            pltpu.VMEM((G, CHUNK_W), jnp.float32),
            pltpu.VMEM((KSUB, BATCH), jnp.int32),
            pltpu.VMEM((KSUB, BATCH), jnp.int32),
            pltpu.SemaphoreType.DMA,
        ],
        compiler_params=pltpu.CompilerParams(use_tc_tiling_on_sc=False),
    )
    return fn(h4, sidx, didx, zrows)


def _chunked(h):
    return h.reshape(N, NCHUNK, CHUNK_W).transpose(1, 0, 2).reshape(NCHUNK * N, CHUNK_W)


def _unchunked(h4):
    return h4.reshape(NCHUNK, N, CHUNK_W).transpose(1, 0, 2).reshape(N, H)


BN = 2000
NBLK = N // BN


def _pre_body(pos, x, pw1, pb1, pw2, pb2, wx, wpe, pjb, pjg, pjbb, rw, rb,
              h_out, hres_out):
    pe = jax.nn.relu(jnp.dot(pos[...], pw1[...],
                             preferred_element_type=jnp.float32) + pb1[...])
    pe = jax.nn.relu(jnp.dot(pe, pw2[...],
                             preferred_element_type=jnp.float32) + pb2[...])
    hh = (jnp.dot(x[...], wx[...], preferred_element_type=jnp.float32)
          + jnp.dot(pe, wpe[...], preferred_element_type=jnp.float32)
          + pjb[...])
    hh = jax.nn.relu(pjg[...] * hh * _BN_S + pjbb[...])
    h_out[...] = hh
    hres_out[...] = jnp.dot(hh, rw[...], preferred_element_type=jnp.float32) + rb[...]


def _pre(x, positions, p):
    row = lambda i: (i, 0)
    full = lambda i: (0, 0)
    return pl.pallas_call(
        _pre_body,
        grid=(NBLK,),
        in_specs=[
            pl.BlockSpec((BN, 3), row),
            pl.BlockSpec((BN, 6), row),
            pl.BlockSpec((3, 16), full),
            pl.BlockSpec((1, 16), full),
            pl.BlockSpec((16, 16), full),
            pl.BlockSpec((1, 16), full),
            pl.BlockSpec((6, H), full),
            pl.BlockSpec((16, H), full),
            pl.BlockSpec((1, H), full),
            pl.BlockSpec((1, H), full),
            pl.BlockSpec((1, H), full),
            pl.BlockSpec((H, H), full),
            pl.BlockSpec((1, H), full),
        ],
        out_specs=[
            pl.BlockSpec((BN, H), row),
            pl.BlockSpec((BN, H), row),
        ],
        out_shape=[
            jax.ShapeDtypeStruct((N, H), jnp.float32),
            jax.ShapeDtypeStruct((N, H), jnp.float32),
        ],
        compiler_params=pltpu.CompilerParams(dimension_semantics=("arbitrary",)),
    )(positions, x,
      p['pos_W1'], p['pos_b1'][None, :], p['pos_W2'], p['pos_b2'][None, :],
      p['proj_W'][:6], p['proj_W'][6:], p['proj_b'][None, :],
      p['proj_g'][None, :], p['proj_bb'][None, :],
      p['res_W'], p['res_b'][None, :])


def _layer_body(first, fused, h, agg, w1, b1, w2, b2, gg, gbb, bg, bb, *rest):
    if first:
        hres, rest = rest[0], rest[1:]
    if fused:
        fw, fb, h_out, f_out = rest
    else:
        (h_out,) = rest
    t = h[...] + agg[...]
    t = jax.nn.relu(jnp.dot(t, w1[...], preferred_element_type=jnp.float32) + b1[...])
    t = jax.nn.relu(jnp.dot(t, w2[...], preferred_element_type=jnp.float32) + b2[...])
    t = gg[...] * t * _BN_S + gbb[...]
    if first:
        t = t + hres[...]
    t = bg[...] * t * _BN_S + bb[...]
    hn = jax.nn.relu(t)
    h_out[...] = hn
    if fused:
        f_out[...] = jax.nn.relu(
            jnp.dot(hn, fw[...], preferred_element_type=jnp.float32) + fb[...])


def _layer(i, h, agg, hres, p):
    first = i == 0
    fused = i >= NL - FL
    row = lambda g: (g, 0)
    full = lambda g: (0, 0)
    in_specs = [
        pl.BlockSpec((BN, H), row),
        pl.BlockSpec((BN, H), row),
        pl.BlockSpec((H, H), full),
        pl.BlockSpec((1, H), full),
        pl.BlockSpec((H, H), full),
        pl.BlockSpec((1, H), full),
        pl.BlockSpec((1, H), full),
        pl.BlockSpec((1, H), full),
        pl.BlockSpec((1, H), full),
        pl.BlockSpec((1, H), full),
    ]
    args = [h, agg,
            p['gin_W1'][i], p['gin_b1'][i][None, :],
            p['gin_W2'][i], p['gin_b2'][i][None, :],
            p['gin_g'][i][None, :], p['gin_bb'][i][None, :],
            p['bn_g'][i][None, :], p['bn_b'][i][None, :]]
    if first:
        in_specs.append(pl.BlockSpec((BN, H), row))
        args.append(hres)
    out_specs = [pl.BlockSpec((BN, H), row)]
    out_shape = [jax.ShapeDtypeStruct((N, H), jnp.float32)]
    if fused:
        j = i - (NL - FL)
        in_specs.append(pl.BlockSpec((H, H // 2), full))
        in_specs.append(pl.BlockSpec((1, H // 2), full))
        args.append(p['fus_W'][j])
        args.append(p['fus_b'][j][None, :])
        out_specs.append(pl.BlockSpec((BN, H // 2), row))
        out_shape.append(jax.ShapeDtypeStruct((N, H // 2), jnp.float32))
    res = pl.pallas_call(
        functools.partial(_layer_body, first, fused),
        grid=(NBLK,),
        in_specs=in_specs,
        out_specs=out_specs,
        out_shape=out_shape,
        compiler_params=pltpu.CompilerParams(dimension_semantics=("arbitrary",)),
    )(*args)
    return res if fused else (res[0], None)


def _attn_body(f0, f1, f2, w1, b1, w2, b2, xf_out, a_out, amax_out, am):
    i = pl.program_id(0)
    xf = jnp.concatenate([f0[...], f1[...], f2[...]], axis=1)
    xf_out[...] = xf
    a = jnp.tanh(jnp.dot(xf, w1[...], preferred_element_type=jnp.float32) + b1[...])
    a = jnp.dot(a, w2[...], preferred_element_type=jnp.float32) + b2[...]
    a_out[...] = a
    m = jnp.max(a)

    @pl.when(i == 0)
    def _():
        am[0, 0] = m

    @pl.when(i > 0)
    def _():
        am[0, 0] = jnp.maximum(am[0, 0], m)

    @pl.when(i == NBLK - 1)
    def _():
        amax_out[...] = jnp.full((1, 1), am[0, 0], jnp.float32)


def _attn(f0, f1, f2, p):
    row = lambda g: (g, 0)
    full = lambda g: (0, 0)
    return pl.pallas_call(
        _attn_body,
        grid=(NBLK,),
        in_specs=[
            pl.BlockSpec((BN, H // 2), row),
            pl.BlockSpec((BN, H // 2), row),
            pl.BlockSpec((BN, H // 2), row),
            pl.BlockSpec((D, H), full),
            pl.BlockSpec((1, H), full),
            pl.BlockSpec((H, 1), full),
            pl.BlockSpec((1, 1), full),
        ],
        out_specs=[
            pl.BlockSpec((BN, D), row),
            pl.BlockSpec((BN, 1), row),
            pl.BlockSpec((1, 1), full),
        ],
        out_shape=[
            jax.ShapeDtypeStruct((N, D), jnp.float32),
            jax.ShapeDtypeStruct((N, 1), jnp.float32),
            jax.ShapeDtypeStruct((1, 1), jnp.float32),
        ],
        scratch_shapes=[pltpu.SMEM((1, 1), jnp.float32)],
        compiler_params=pltpu.CompilerParams(dimension_semantics=("arbitrary",)),
    )(f0, f1, f2, p['attn_W1'], p['attn_b1'][None, :],
      p['attn_W2'], p['attn_b2'][None, :])


def _pool_body(xf, a, amax, bid, ow1, ob1, ow2, ob2, o_out,
               s_wsum, s_sum, s_max, s_cnt, s_es):
    i = pl.program_id(0)

    @pl.when(i == 0)
    def _():
        s_wsum[...] = jnp.zeros_like(s_wsum)
        s_sum[...] = jnp.zeros_like(s_sum)
        s_max[...] = jnp.full_like(s_max, -jnp.inf)
        s_cnt[...] = jnp.zeros_like(s_cnt)
        s_es[0, 0] = 0.0

    xfv = xf[...]
    ids = bid[...]
    e = jnp.exp(a[...] - amax[0, 0])
    oh = (ids == lax.broadcasted_iota(jnp.int32, (1, B), 1)).astype(jnp.float32)
    ct = (((0,), (0,)), ((), ()))
    s_wsum[...] += lax.dot_general(oh, xfv * e, ct,
                                   preferred_element_type=jnp.float32)
    s_sum[...] += lax.dot_general(oh, xfv, ct,
                                  preferred_element_type=jnp.float32)
    s_cnt[...] += jnp.broadcast_to(jnp.sum(oh, axis=0)[:, None], s_cnt.shape)
    s_es[0, 0] += jnp.sum(e)
    bmin = jnp.min(ids)
    bmax = jnp.max(ids)
    for b in range(B):
        @pl.when((bmin <= b) & (b <= bmax))
        def _():
            cur = jnp.max(jnp.where(ids == b, xfv, -jnp.inf), axis=0)
            s_max[pl.ds(b, 1), :] = jnp.maximum(s_max[pl.ds(b, 1), :], cur[None, :])

    @pl.when(i == NBLK - 1)
    def _():
        cnt = jnp.maximum(s_cnt[...][:, :1], 1.0)
        wsum = s_wsum[...] / (s_es[0, 0] * cnt)
        gmean = s_sum[...] / cnt
        pooled = jnp.concatenate([wsum, gmean, s_max[...]], axis=1)
        o = jax.nn.relu(jnp.dot(pooled, ow1[...],
                                preferred_element_type=jnp.float32) + ob1[...])
        o_out[...] = jax.nn.relu(jnp.dot(o, ow2[...],
                                         preferred_element_type=jnp.float32) + ob2[...])


def _pool(xf, a, amax, batch2, p):
    row = lambda g: (g, 0)
    full = lambda g: (0, 0)
    return pl.pallas_call(
        _pool_body,
        grid=(NBLK,),
        in_specs=[
            pl.BlockSpec((BN, D), row),
            pl.BlockSpec((BN, 1), row),
            pl.BlockSpec(memory_space=pltpu.SMEM),
            pl.BlockSpec((BN, 1), row),
            pl.BlockSpec((3 * D, 2 * H), full),
            pl.BlockSpec((1, 2 * H), full),
            pl.BlockSpec((2 * H, OUT), full),
            pl.BlockSpec((1, OUT), full),
        ],
        out_specs=pl.BlockSpec((B, OUT), full),
        out_shape=jax.ShapeDtypeStruct((B, OUT), jnp.float32),
        scratch_shapes=[
            pltpu.VMEM((B, D), jnp.float32),
            pltpu.VMEM((B, D), jnp.float32),
            pltpu.VMEM((B, D), jnp.float32),
            pltpu.VMEM((B, 128), jnp.float32),
            pltpu.SMEM((1, 1), jnp.float32),
        ],
        compiler_params=pltpu.CompilerParams(dimension_semantics=("arbitrary",)),
    )(xf, a, amax, batch2,
      p['out_W1'], p['out_b1'][None, :], p['out_W2'], p['out_b2'][None, :])


def kernel(x, edge_index, positions, batch, params):
    p = params
    src = edge_index[0].astype(jnp.int32)
    dst = edge_index[1].astype(jnp.int32)
    offs = (jnp.arange(NCHUNK, dtype=jnp.int32) * N)[:, None]
    sidx = (src[None, :] + offs).reshape(NCHUNK * E // BATCH, BATCH)
    didx = dst.reshape(E // BATCH, BATCH)
    zrows = jnp.zeros((STRIPE, CHUNK_W), jnp.float32)
    batch2 = batch.astype(jnp.int32).reshape(N, 1)

    h, hres = _pre(x, positions, p)
    feats = []
    for i in range(NL):
        agg = _unchunked(_sc_segment_sum(_chunked(h), sidx, didx, zrows))
        h, f = _layer(i, h, agg, hres, p)
        if f is not None:
            feats.append(f)
    xf, a, amax = _attn(feats[0], feats[1], feats[2], p)
    return _pool(xf, a, amax, batch2, p)


# speedup vs baseline: 3.8336x; 1.0547x over previous
import functools

import jax
import jax.numpy as jnp
from jax import lax
from jax.experimental import pallas as pl
from jax.experimental.pallas import tpu as pltpu
from jax.experimental.pallas import tpu_sc as plsc

N = 50000
E = 800000
B = 64
H = 128
OUT = 256
NL = 5
FL = 3
D = (H // 2) * FL

_BN_S = float((1.0 + 1e-5) ** -0.5)
NEG_BIG = float(jnp.finfo(jnp.float32).min)

NSUB = 16
NCORE = 2
CHUNK_W = 32
NCHUNK = H // CHUNK_W
BATCH = 125
KSUB = 2
G = KSUB * BATCH
EP = E // NSUB
NSTEP = EP // G
STRIPE = 3200
STRIPE_LAST = N - STRIPE * (NSUB - 1)


def _agg_body(h4, sidx, didx, zrows, out, acc,
              rows0, rows1, sv0, sv1, dv0, dv1, sem0, sem1):
    cid = lax.axis_index("c")
    sid = lax.axis_index("s")
    last = NSUB - 1
    bufs = ((rows0, sv0, dv0, sem0), (rows1, sv1, dv1, sem1))
    for step in range(NCHUNK // NCORE):
        c = cid + NCORE * step
        base_s = c * (E // BATCH) + sid * (EP // BATCH)
        base_d = sid * (EP // BATCH)

        @pl.when(sid < last)
        def _():
            pltpu.sync_copy(zrows, acc.at[pl.ds(sid * STRIPE, STRIPE)])

        @pl.when(sid == last)
        def _():
            pltpu.sync_copy(zrows.at[pl.ds(0, STRIPE_LAST)],
                            acc.at[pl.ds(sid * STRIPE, STRIPE_LAST)])

        plsc.subcore_barrier()

        def prefetch(i, buf):
            rows, sv, dv, sem = buf
            pltpu.sync_copy(sidx.at[pl.ds(base_s + i * KSUB, KSUB)], sv)
            pltpu.sync_copy(didx.at[pl.ds(base_d + i * KSUB, KSUB)], dv)
            for j in range(KSUB):
                pltpu.async_copy(h4.at[sv.at[j]],
                                 rows.at[pl.ds(j * BATCH, BATCH)], sem)

        def drain_scatter(buf):
            rows, sv, dv, sem = buf
            for j in range(KSUB):
                pltpu.make_async_copy(h4.at[sv.at[j]],
                                      rows.at[pl.ds(j * BATCH, BATCH)],
                                      sem).wait()
            for j in range(KSUB):
                pltpu.sync_copy(rows.at[pl.ds(j * BATCH, BATCH)],
                                acc.at[dv.at[j]], add=True)

        prefetch(0, bufs[0])

        def body(k, carry):
            prefetch(2 * k + 1, bufs[1])
            drain_scatter(bufs[0])

            @pl.when(k < NSTEP // 2 - 1)
            def _():
                prefetch(2 * k + 2, bufs[0])

            drain_scatter(bufs[1])
            return carry

        lax.fori_loop(0, NSTEP // 2, body, None)
        plsc.subcore_barrier()

        @pl.when(sid < last)
        def _():
            pltpu.sync_copy(acc.at[pl.ds(sid * STRIPE, STRIPE)],
                            out.at[pl.ds(c * N + sid * STRIPE, STRIPE)])

        @pl.when(sid == last)
        def _():
            pltpu.sync_copy(acc.at[pl.ds(sid * STRIPE, STRIPE_LAST)],
                            out.at[pl.ds(c * N + sid * STRIPE, STRIPE_LAST)])

        plsc.subcore_barrier()


def _sc_segment_sum(h4, sidx, didx, zrows):
    mesh = plsc.VectorSubcoreMesh(core_axis_name="c", subcore_axis_name="s")
    fn = pl.kernel(
        _agg_body,
        out_type=jax.ShapeDtypeStruct((NCHUNK * N, CHUNK_W), jnp.float32),
        mesh=mesh,
        scratch_types=[
            pltpu.VMEM_SHARED((N, CHUNK_W), jnp.float32),
            pltpu.VMEM((G, CHUNK_W), jnp.float32),
            pltpu.VMEM((G, CHUNK_W), jnp.float32),
            pltpu.VMEM((KSUB, BATCH), jnp.int32),
            pltpu.VMEM((KSUB, BATCH), jnp.int32),
            pltpu.VMEM((KSUB, BATCH), jnp.int32),
            pltpu.VMEM((KSUB, BATCH), jnp.int32),
            pltpu.SemaphoreType.DMA,
            pltpu.SemaphoreType.DMA,
        ],
        compiler_params=pltpu.CompilerParams(use_tc_tiling_on_sc=False),
    )
    return fn(h4, sidx, didx, zrows)


def _chunked(h):
    return h.reshape(N, NCHUNK, CHUNK_W).transpose(1, 0, 2).reshape(NCHUNK * N, CHUNK_W)


def _unchunked(h4):
    return h4.reshape(NCHUNK, N, CHUNK_W).transpose(1, 0, 2).reshape(N, H)


BN = 2000
NBLK = N // BN


def _pre_body(pos, x, pw1, pb1, pw2, pb2, wx, wpe, pjb, pjg, pjbb, rw, rb,
              h_out, hres_out):
    pe = jax.nn.relu(jnp.dot(pos[...], pw1[...],
                             preferred_element_type=jnp.float32) + pb1[...])
    pe = jax.nn.relu(jnp.dot(pe, pw2[...],
                             preferred_element_type=jnp.float32) + pb2[...])
    hh = (jnp.dot(x[...], wx[...], preferred_element_type=jnp.float32)
          + jnp.dot(pe, wpe[...], preferred_element_type=jnp.float32)
          + pjb[...])
    hh = jax.nn.relu(pjg[...] * hh * _BN_S + pjbb[...])
    h_out[...] = hh
    hres_out[...] = jnp.dot(hh, rw[...], preferred_element_type=jnp.float32) + rb[...]


def _pre(x, positions, p):
    row = lambda i: (i, 0)
    full = lambda i: (0, 0)
    return pl.pallas_call(
        _pre_body,
        grid=(NBLK,),
        in_specs=[
            pl.BlockSpec((BN, 3), row),
            pl.BlockSpec((BN, 6), row),
            pl.BlockSpec((3, 16), full),
            pl.BlockSpec((1, 16), full),
            pl.BlockSpec((16, 16), full),
            pl.BlockSpec((1, 16), full),
            pl.BlockSpec((6, H), full),
            pl.BlockSpec((16, H), full),
            pl.BlockSpec((1, H), full),
            pl.BlockSpec((1, H), full),
            pl.BlockSpec((1, H), full),
            pl.BlockSpec((H, H), full),
            pl.BlockSpec((1, H), full),
        ],
        out_specs=[
            pl.BlockSpec((BN, H), row),
            pl.BlockSpec((BN, H), row),
        ],
        out_shape=[
            jax.ShapeDtypeStruct((N, H), jnp.float32),
            jax.ShapeDtypeStruct((N, H), jnp.float32),
        ],
        compiler_params=pltpu.CompilerParams(dimension_semantics=("arbitrary",)),
    )(positions, x,
      p['pos_W1'], p['pos_b1'][None, :], p['pos_W2'], p['pos_b2'][None, :],
      p['proj_W'][:6], p['proj_W'][6:], p['proj_b'][None, :],
      p['proj_g'][None, :], p['proj_bb'][None, :],
      p['res_W'], p['res_b'][None, :])


def _layer_body(first, fused, h, agg, w1, b1, w2, b2, gg, gbb, bg, bb, *rest):
    if first:
        hres, rest = rest[0], rest[1:]
    if fused:
        fw, fb, h_out, f_out = rest
    else:
        (h_out,) = rest
    t = h[...] + agg[...]
    t = jax.nn.relu(jnp.dot(t, w1[...], preferred_element_type=jnp.float32) + b1[...])
    t = jax.nn.relu(jnp.dot(t, w2[...], preferred_element_type=jnp.float32) + b2[...])
    t = gg[...] * t * _BN_S + gbb[...]
    if first:
        t = t + hres[...]
    t = bg[...] * t * _BN_S + bb[...]
    hn = jax.nn.relu(t)
    h_out[...] = hn
    if fused:
        f_out[...] = jax.nn.relu(
            jnp.dot(hn, fw[...], preferred_element_type=jnp.float32) + fb[...])


def _layer(i, h, agg, hres, p):
    first = i == 0
    fused = i >= NL - FL
    row = lambda g: (g, 0)
    full = lambda g: (0, 0)
    in_specs = [
        pl.BlockSpec((BN, H), row),
        pl.BlockSpec((BN, H), row),
        pl.BlockSpec((H, H), full),
        pl.BlockSpec((1, H), full),
        pl.BlockSpec((H, H), full),
        pl.BlockSpec((1, H), full),
        pl.BlockSpec((1, H), full),
        pl.BlockSpec((1, H), full),
        pl.BlockSpec((1, H), full),
        pl.BlockSpec((1, H), full),
    ]
    args = [h, agg,
            p['gin_W1'][i], p['gin_b1'][i][None, :],
            p['gin_W2'][i], p['gin_b2'][i][None, :],
            p['gin_g'][i][None, :], p['gin_bb'][i][None, :],
            p['bn_g'][i][None, :], p['bn_b'][i][None, :]]
    if first:
        in_specs.append(pl.BlockSpec((BN, H), row))
        args.append(hres)
    out_specs = [pl.BlockSpec((BN, H), row)]
    out_shape = [jax.ShapeDtypeStruct((N, H), jnp.float32)]
    if fused:
        j = i - (NL - FL)
        in_specs.append(pl.BlockSpec((H, H // 2), full))
        in_specs.append(pl.BlockSpec((1, H // 2), full))
        args.append(p['fus_W'][j])
        args.append(p['fus_b'][j][None, :])
        out_specs.append(pl.BlockSpec((BN, H // 2), row))
        out_shape.append(jax.ShapeDtypeStruct((N, H // 2), jnp.float32))
    res = pl.pallas_call(
        functools.partial(_layer_body, first, fused),
        grid=(NBLK,),
        in_specs=in_specs,
        out_specs=out_specs,
        out_shape=out_shape,
        compiler_params=pltpu.CompilerParams(dimension_semantics=("arbitrary",)),
    )(*args)
    return res if fused else (res[0], None)


def _attn_body(f0, f1, f2, w1, b1, w2, b2, xf_out, a_out, amax_out, am):
    i = pl.program_id(0)
    xf = jnp.concatenate([f0[...], f1[...], f2[...]], axis=1)
    xf_out[...] = xf
    a = jnp.tanh(jnp.dot(xf, w1[...], preferred_element_type=jnp.float32) + b1[...])
    a = jnp.dot(a, w2[...], preferred_element_type=jnp.float32) + b2[...]
    a_out[...] = a
    m = jnp.max(a)

    @pl.when(i == 0)
    def _():
        am[0, 0] = m

    @pl.when(i > 0)
    def _():
        am[0, 0] = jnp.maximum(am[0, 0], m)

    @pl.when(i == NBLK - 1)
    def _():
        amax_out[...] = jnp.full((1, 1), am[0, 0], jnp.float32)


def _attn(f0, f1, f2, p):
    row = lambda g: (g, 0)
    full = lambda g: (0, 0)
    return pl.pallas_call(
        _attn_body,
        grid=(NBLK,),
        in_specs=[
            pl.BlockSpec((BN, H // 2), row),
            pl.BlockSpec((BN, H // 2), row),
            pl.BlockSpec((BN, H // 2), row),
            pl.BlockSpec((D, H), full),
            pl.BlockSpec((1, H), full),
            pl.BlockSpec((H, 1), full),
            pl.BlockSpec((1, 1), full),
        ],
        out_specs=[
            pl.BlockSpec((BN, D), row),
            pl.BlockSpec((BN, 1), row),
            pl.BlockSpec((1, 1), full),
        ],
        out_shape=[
            jax.ShapeDtypeStruct((N, D), jnp.float32),
            jax.ShapeDtypeStruct((N, 1), jnp.float32),
            jax.ShapeDtypeStruct((1, 1), jnp.float32),
        ],
        scratch_shapes=[pltpu.SMEM((1, 1), jnp.float32)],
        compiler_params=pltpu.CompilerParams(dimension_semantics=("arbitrary",)),
    )(f0, f1, f2, p['attn_W1'], p['attn_b1'][None, :],
      p['attn_W2'], p['attn_b2'][None, :])


def _pool_body(xf, a, amax, bid, ow1, ob1, ow2, ob2, o_out,
               s_wsum, s_sum, s_max, s_cnt, s_es):
    i = pl.program_id(0)

    @pl.when(i == 0)
    def _():
        s_wsum[...] = jnp.zeros_like(s_wsum)
        s_sum[...] = jnp.zeros_like(s_sum)
        s_max[...] = jnp.full_like(s_max, -jnp.inf)
        s_cnt[...] = jnp.zeros_like(s_cnt)
        s_es[0, 0] = 0.0

    xfv = xf[...]
    ids = bid[...]
    e = jnp.exp(a[...] - amax[0, 0])
    oh = (ids == lax.broadcasted_iota(jnp.int32, (1, B), 1)).astype(jnp.float32)
    ct = (((0,), (0,)), ((), ()))
    s_wsum[...] += lax.dot_general(oh, xfv * e, ct,
                                   preferred_element_type=jnp.float32)
    s_sum[...] += lax.dot_general(oh, xfv, ct,
                                  preferred_element_type=jnp.float32)
    s_cnt[...] += jnp.broadcast_to(jnp.sum(oh, axis=0)[:, None], s_cnt.shape)
    s_es[0, 0] += jnp.sum(e)
    bmin = jnp.min(ids)
    bmax = jnp.max(ids)
    for b in range(B):
        @pl.when((bmin <= b) & (b <= bmax))
        def _():
            cur = jnp.max(jnp.where(ids == b, xfv, -jnp.inf), axis=0)
            s_max[pl.ds(b, 1), :] = jnp.maximum(s_max[pl.ds(b, 1), :], cur[None, :])

    @pl.when(i == NBLK - 1)
    def _():
        cnt = jnp.maximum(s_cnt[...][:, :1], 1.0)
        wsum = s_wsum[...] / (s_es[0, 0] * cnt)
        gmean = s_sum[...] / cnt
        pooled = jnp.concatenate([wsum, gmean, s_max[...]], axis=1)
        o = jax.nn.relu(jnp.dot(pooled, ow1[...],
                                preferred_element_type=jnp.float32) + ob1[...])
        o_out[...] = jax.nn.relu(jnp.dot(o, ow2[...],
                                         preferred_element_type=jnp.float32) + ob2[...])


def _pool(xf, a, amax, batch2, p):
    row = lambda g: (g, 0)
    full = lambda g: (0, 0)
    return pl.pallas_call(
        _pool_body,
        grid=(NBLK,),
        in_specs=[
            pl.BlockSpec((BN, D), row),
            pl.BlockSpec((BN, 1), row),
            pl.BlockSpec(memory_space=pltpu.SMEM),
            pl.BlockSpec((BN, 1), row),
            pl.BlockSpec((3 * D, 2 * H), full),
            pl.BlockSpec((1, 2 * H), full),
            pl.BlockSpec((2 * H, OUT), full),
            pl.BlockSpec((1, OUT), full),
        ],
        out_specs=pl.BlockSpec((B, OUT), full),
        out_shape=jax.ShapeDtypeStruct((B, OUT), jnp.float32),
        scratch_shapes=[
            pltpu.VMEM((B, D), jnp.float32),
            pltpu.VMEM((B, D), jnp.float32),
            pltpu.VMEM((B, D), jnp.float32),
            pltpu.VMEM((B, 128), jnp.float32),
            pltpu.SMEM((1, 1), jnp.float32),
        ],
        compiler_params=pltpu.CompilerParams(dimension_semantics=("arbitrary",)),
    )(xf, a, amax, batch2,
      p['out_W1'], p['out_b1'][None, :], p['out_W2'], p['out_b2'][None, :])


def kernel(x, edge_index, positions, batch, params):
    p = params
    src = edge_index[0].astype(jnp.int32)
    dst = edge_index[1].astype(jnp.int32)
    offs = (jnp.arange(NCHUNK, dtype=jnp.int32) * N)[:, None]
    sidx = (src[None, :] + offs).reshape(NCHUNK * E // BATCH, BATCH)
    didx = dst.reshape(E // BATCH, BATCH)
    zrows = jnp.zeros((STRIPE, CHUNK_W), jnp.float32)
    batch2 = batch.astype(jnp.int32).reshape(N, 1)

    h, hres = _pre(x, positions, p)
    feats = []
    for i in range(NL):
        agg = _unchunked(_sc_segment_sum(_chunked(h), sidx, didx, zrows))
        h, f = _layer(i, h, agg, hres, p)
        if f is not None:
            feats.append(f)
    xf, a, amax = _attn(feats[0], feats[1], feats[2], p)
    return _pool(xf, a, amax, batch2, p)


# speedup vs baseline: 5.1801x; 1.3512x over previous
import functools

import jax
import jax.numpy as jnp
from jax import lax
from jax.experimental import pallas as pl
from jax.experimental.pallas import tpu as pltpu
from jax.experimental.pallas import tpu_sc as plsc

N = 50000
E = 800000
B = 64
H = 128
OUT = 256
NL = 5
FL = 3
D = (H // 2) * FL

_BN_S = float((1.0 + 1e-5) ** -0.5)
NEG_BIG = float(jnp.finfo(jnp.float32).min)

NSUB = 16
NCORE = 2
CHUNK_W = 32
NCHUNK = H // CHUNK_W
BATCH = 125
KSUB = 2
G = KSUB * BATCH
EP = E // NSUB
NSTEP = EP // G
TSTEP = 20
NIBLK = NSTEP // TSTEP
STRIPE = 3200
STRIPE_LAST = N - STRIPE * (NSUB - 1)


def _agg_body(h4, sidx, didx, zrows, out, acc,
              rows0, rows1, svb, dvb, sem0, sem1):
    cid = lax.axis_index("c")
    sid = lax.axis_index("s")
    last = NSUB - 1
    bufs = ((rows0, sem0), (rows1, sem1))
    for step in range(NCHUNK // NCORE):
        c = cid + NCORE * step
        base_s = c * (E // BATCH) + sid * (EP // BATCH)
        base_d = sid * (EP // BATCH)

        @pl.when(sid < last)
        def _():
            pltpu.sync_copy(zrows, acc.at[pl.ds(sid * STRIPE, STRIPE)])

        @pl.when(sid == last)
        def _():
            pltpu.sync_copy(zrows.at[pl.ds(0, STRIPE_LAST)],
                            acc.at[pl.ds(sid * STRIPE, STRIPE_LAST)])

        plsc.subcore_barrier()

        def fire(t, buf):
            rows, sem = buf
            for j in range(KSUB):
                pltpu.async_copy(h4.at[svb.at[t * KSUB + j]],
                                 rows.at[pl.ds(j * BATCH, BATCH)], sem)

        def drain_scatter(t, buf):
            rows, sem = buf
            for j in range(KSUB):
                pltpu.make_async_copy(h4.at[svb.at[t * KSUB + j]],
                                      rows.at[pl.ds(j * BATCH, BATCH)],
                                      sem).wait()
            for j in range(KSUB):
                pltpu.sync_copy(rows.at[pl.ds(j * BATCH, BATCH)],
                                acc.at[dvb.at[t * KSUB + j]], add=True)

        def block_body(b, carry):
            pltpu.sync_copy(
                sidx.at[pl.ds(base_s + b * TSTEP * KSUB, TSTEP * KSUB)], svb)
            pltpu.sync_copy(
                didx.at[pl.ds(base_d + b * TSTEP * KSUB, TSTEP * KSUB)], dvb)
            fire(0, bufs[0])

            def pair(u, carry2):
                fire(2 * u + 1, bufs[1])
                drain_scatter(2 * u, bufs[0])

                @pl.when(u < TSTEP // 2 - 1)
                def _():
                    fire(2 * u + 2, bufs[0])

                drain_scatter(2 * u + 1, bufs[1])
                return carry2

            lax.fori_loop(0, TSTEP // 2, pair, None)
            return carry

        lax.fori_loop(0, NIBLK, block_body, None)
        plsc.subcore_barrier()

        @pl.when(sid < last)
        def _():
            pltpu.sync_copy(acc.at[pl.ds(sid * STRIPE, STRIPE)],
                            out.at[pl.ds(c * N + sid * STRIPE, STRIPE)])

        @pl.when(sid == last)
        def _():
            pltpu.sync_copy(acc.at[pl.ds(sid * STRIPE, STRIPE_LAST)],
                            out.at[pl.ds(c * N + sid * STRIPE, STRIPE_LAST)])

        plsc.subcore_barrier()


def _sc_segment_sum(h4, sidx, didx, zrows):
    mesh = plsc.VectorSubcoreMesh(core_axis_name="c", subcore_axis_name="s")
    fn = pl.kernel(
        _agg_body,
        out_type=jax.ShapeDtypeStruct((NCHUNK * N, CHUNK_W), jnp.float32),
        mesh=mesh,
        scratch_types=[
            pltpu.VMEM_SHARED((N, CHUNK_W), jnp.float32),
            pltpu.VMEM((G, CHUNK_W), jnp.float32),
            pltpu.VMEM((G, CHUNK_W), jnp.float32),
            pltpu.VMEM((TSTEP * KSUB, BATCH), jnp.int32),
            pltpu.VMEM((TSTEP * KSUB, BATCH), jnp.int32),
            pltpu.SemaphoreType.DMA,
            pltpu.SemaphoreType.DMA,
        ],
        compiler_params=pltpu.CompilerParams(use_tc_tiling_on_sc=False),
    )
    return fn(h4, sidx, didx, zrows)


def _chunked(h):
    return h.reshape(N, NCHUNK, CHUNK_W).transpose(1, 0, 2).reshape(NCHUNK * N, CHUNK_W)


def _unchunked(h4):
    return h4.reshape(NCHUNK, N, CHUNK_W).transpose(1, 0, 2).reshape(N, H)


BN = 2000
NBLK = N // BN


def _pre_body(pos, x, pw1, pb1, pw2, pb2, wx, wpe, pjb, pjg, pjbb, rw, rb,
              h_out, hres_out):
    pe = jax.nn.relu(jnp.dot(pos[...], pw1[...],
                             preferred_element_type=jnp.float32) + pb1[...])
    pe = jax.nn.relu(jnp.dot(pe, pw2[...],
                             preferred_element_type=jnp.float32) + pb2[...])
    hh = (jnp.dot(x[...], wx[...], preferred_element_type=jnp.float32)
          + jnp.dot(pe, wpe[...], preferred_element_type=jnp.float32)
          + pjb[...])
    hh = jax.nn.relu(pjg[...] * hh * _BN_S + pjbb[...])
    h_out[...] = hh
    hres_out[...] = jnp.dot(hh, rw[...], preferred_element_type=jnp.float32) + rb[...]


def _pre(x, positions, p):
    row = lambda i: (i, 0)
    full = lambda i: (0, 0)
    return pl.pallas_call(
        _pre_body,
        grid=(NBLK,),
        in_specs=[
            pl.BlockSpec((BN, 3), row),
            pl.BlockSpec((BN, 6), row),
            pl.BlockSpec((3, 16), full),
            pl.BlockSpec((1, 16), full),
            pl.BlockSpec((16, 16), full),
            pl.BlockSpec((1, 16), full),
            pl.BlockSpec((6, H), full),
            pl.BlockSpec((16, H), full),
            pl.BlockSpec((1, H), full),
            pl.BlockSpec((1, H), full),
            pl.BlockSpec((1, H), full),
            pl.BlockSpec((H, H), full),
            pl.BlockSpec((1, H), full),
        ],
        out_specs=[
            pl.BlockSpec((BN, H), row),
            pl.BlockSpec((BN, H), row),
        ],
        out_shape=[
            jax.ShapeDtypeStruct((N, H), jnp.float32),
            jax.ShapeDtypeStruct((N, H), jnp.float32),
        ],
        compiler_params=pltpu.CompilerParams(dimension_semantics=("arbitrary",)),
    )(positions, x,
      p['pos_W1'], p['pos_b1'][None, :], p['pos_W2'], p['pos_b2'][None, :],
      p['proj_W'][:6], p['proj_W'][6:], p['proj_b'][None, :],
      p['proj_g'][None, :], p['proj_bb'][None, :],
      p['res_W'], p['res_b'][None, :])


def _layer_body(first, fused, h, agg, w1, b1, w2, b2, gg, gbb, bg, bb, *rest):
    if first:
        hres, rest = rest[0], rest[1:]
    if fused:
        fw, fb, h_out, f_out = rest
    else:
        (h_out,) = rest
    t = h[...] + agg[...]
    t = jax.nn.relu(jnp.dot(t, w1[...], preferred_element_type=jnp.float32) + b1[...])
    t = jax.nn.relu(jnp.dot(t, w2[...], preferred_element_type=jnp.float32) + b2[...])
    t = gg[...] * t * _BN_S + gbb[...]
    if first:
        t = t + hres[...]
    t = bg[...] * t * _BN_S + bb[...]
    hn = jax.nn.relu(t)
    h_out[...] = hn
    if fused:
        f_out[...] = jax.nn.relu(
            jnp.dot(hn, fw[...], preferred_element_type=jnp.float32) + fb[...])


def _layer(i, h, agg, hres, p):
    first = i == 0
    fused = i >= NL - FL
    row = lambda g: (g, 0)
    full = lambda g: (0, 0)
    in_specs = [
        pl.BlockSpec((BN, H), row),
        pl.BlockSpec((BN, H), row),
        pl.BlockSpec((H, H), full),
        pl.BlockSpec((1, H), full),
        pl.BlockSpec((H, H), full),
        pl.BlockSpec((1, H), full),
        pl.BlockSpec((1, H), full),
        pl.BlockSpec((1, H), full),
        pl.BlockSpec((1, H), full),
        pl.BlockSpec((1, H), full),
    ]
    args = [h, agg,
            p['gin_W1'][i], p['gin_b1'][i][None, :],
            p['gin_W2'][i], p['gin_b2'][i][None, :],
            p['gin_g'][i][None, :], p['gin_bb'][i][None, :],
            p['bn_g'][i][None, :], p['bn_b'][i][None, :]]
    if first:
        in_specs.append(pl.BlockSpec((BN, H), row))
        args.append(hres)
    out_specs = [pl.BlockSpec((BN, H), row)]
    out_shape = [jax.ShapeDtypeStruct((N, H), jnp.float32)]
    if fused:
        j = i - (NL - FL)
        in_specs.append(pl.BlockSpec((H, H // 2), full))
        in_specs.append(pl.BlockSpec((1, H // 2), full))
        args.append(p['fus_W'][j])
        args.append(p['fus_b'][j][None, :])
        out_specs.append(pl.BlockSpec((BN, H // 2), row))
        out_shape.append(jax.ShapeDtypeStruct((N, H // 2), jnp.float32))
    res = pl.pallas_call(
        functools.partial(_layer_body, first, fused),
        grid=(NBLK,),
        in_specs=in_specs,
        out_specs=out_specs,
        out_shape=out_shape,
        compiler_params=pltpu.CompilerParams(dimension_semantics=("arbitrary",)),
    )(*args)
    return res if fused else (res[0], None)


def _attn_body(f0, f1, f2, w1, b1, w2, b2, xf_out, a_out, amax_out, am):
    i = pl.program_id(0)
    xf = jnp.concatenate([f0[...], f1[...], f2[...]], axis=1)
    xf_out[...] = xf
    a = jnp.tanh(jnp.dot(xf, w1[...], preferred_element_type=jnp.float32) + b1[...])
    a = jnp.dot(a, w2[...], preferred_element_type=jnp.float32) + b2[...]
    a_out[...] = a
    m = jnp.max(a)

    @pl.when(i == 0)
    def _():
        am[0, 0] = m

    @pl.when(i > 0)
    def _():
        am[0, 0] = jnp.maximum(am[0, 0], m)

    @pl.when(i == NBLK - 1)
    def _():
        amax_out[...] = jnp.full((1, 1), am[0, 0], jnp.float32)


def _attn(f0, f1, f2, p):
    row = lambda g: (g, 0)
    full = lambda g: (0, 0)
    return pl.pallas_call(
        _attn_body,
        grid=(NBLK,),
        in_specs=[
            pl.BlockSpec((BN, H // 2), row),
            pl.BlockSpec((BN, H // 2), row),
            pl.BlockSpec((BN, H // 2), row),
            pl.BlockSpec((D, H), full),
            pl.BlockSpec((1, H), full),
            pl.BlockSpec((H, 1), full),
            pl.BlockSpec((1, 1), full),
        ],
        out_specs=[
            pl.BlockSpec((BN, D), row),
            pl.BlockSpec((BN, 1), row),
            pl.BlockSpec((1, 1), full),
        ],
        out_shape=[
            jax.ShapeDtypeStruct((N, D), jnp.float32),
            jax.ShapeDtypeStruct((N, 1), jnp.float32),
            jax.ShapeDtypeStruct((1, 1), jnp.float32),
        ],
        scratch_shapes=[pltpu.SMEM((1, 1), jnp.float32)],
        compiler_params=pltpu.CompilerParams(dimension_semantics=("arbitrary",)),
    )(f0, f1, f2, p['attn_W1'], p['attn_b1'][None, :],
      p['attn_W2'], p['attn_b2'][None, :])


def _pool_body(xf, a, amax, bid, ow1, ob1, ow2, ob2, o_out,
               s_wsum, s_sum, s_max, s_cnt, s_es):
    i = pl.program_id(0)

    @pl.when(i == 0)
    def _():
        s_wsum[...] = jnp.zeros_like(s_wsum)
        s_sum[...] = jnp.zeros_like(s_sum)
        s_max[...] = jnp.full_like(s_max, -jnp.inf)
        s_cnt[...] = jnp.zeros_like(s_cnt)
        s_es[0, 0] = 0.0

    xfv = xf[...]
    ids = bid[...]
    e = jnp.exp(a[...] - amax[0, 0])
    oh = (ids == lax.broadcasted_iota(jnp.int32, (1, B), 1)).astype(jnp.float32)
    ct = (((0,), (0,)), ((), ()))
    s_wsum[...] += lax.dot_general(oh, xfv * e, ct,
                                   preferred_element_type=jnp.float32)
    s_sum[...] += lax.dot_general(oh, xfv, ct,
                                  preferred_element_type=jnp.float32)
    s_cnt[...] += jnp.broadcast_to(jnp.sum(oh, axis=0)[:, None], s_cnt.shape)
    s_es[0, 0] += jnp.sum(e)
    bmin = jnp.min(ids)
    bmax = jnp.max(ids)
    for b in range(B):
        @pl.when((bmin <= b) & (b <= bmax))
        def _():
            cur = jnp.max(jnp.where(ids == b, xfv, -jnp.inf), axis=0)
            s_max[pl.ds(b, 1), :] = jnp.maximum(s_max[pl.ds(b, 1), :], cur[None, :])

    @pl.when(i == NBLK - 1)
    def _():
        cnt = jnp.maximum(s_cnt[...][:, :1], 1.0)
        wsum = s_wsum[...] / (s_es[0, 0] * cnt)
        gmean = s_sum[...] / cnt
        pooled = jnp.concatenate([wsum, gmean, s_max[...]], axis=1)
        o = jax.nn.relu(jnp.dot(pooled, ow1[...],
                                preferred_element_type=jnp.float32) + ob1[...])
        o_out[...] = jax.nn.relu(jnp.dot(o, ow2[...],
                                         preferred_element_type=jnp.float32) + ob2[...])


def _pool(xf, a, amax, batch2, p):
    row = lambda g: (g, 0)
    full = lambda g: (0, 0)
    return pl.pallas_call(
        _pool_body,
        grid=(NBLK,),
        in_specs=[
            pl.BlockSpec((BN, D), row),
            pl.BlockSpec((BN, 1), row),
            pl.BlockSpec(memory_space=pltpu.SMEM),
            pl.BlockSpec((BN, 1), row),
            pl.BlockSpec((3 * D, 2 * H), full),
            pl.BlockSpec((1, 2 * H), full),
            pl.BlockSpec((2 * H, OUT), full),
            pl.BlockSpec((1, OUT), full),
        ],
        out_specs=pl.BlockSpec((B, OUT), full),
        out_shape=jax.ShapeDtypeStruct((B, OUT), jnp.float32),
        scratch_shapes=[
            pltpu.VMEM((B, D), jnp.float32),
            pltpu.VMEM((B, D), jnp.float32),
            pltpu.VMEM((B, D), jnp.float32),
            pltpu.VMEM((B, 128), jnp.float32),
            pltpu.SMEM((1, 1), jnp.float32),
        ],
        compiler_params=pltpu.CompilerParams(dimension_semantics=("arbitrary",)),
    )(xf, a, amax, batch2,
      p['out_W1'], p['out_b1'][None, :], p['out_W2'], p['out_b2'][None, :])


def kernel(x, edge_index, positions, batch, params):
    p = params
    src = edge_index[0].astype(jnp.int32)
    dst = edge_index[1].astype(jnp.int32)
    offs = (jnp.arange(NCHUNK, dtype=jnp.int32) * N)[:, None]
    sidx = (src[None, :] + offs).reshape(NCHUNK * E // BATCH, BATCH)
    didx = dst.reshape(E // BATCH, BATCH)
    zrows = jnp.zeros((STRIPE, CHUNK_W), jnp.float32)
    batch2 = batch.astype(jnp.int32).reshape(N, 1)

    h, hres = _pre(x, positions, p)
    feats = []
    for i in range(NL):
        agg = _unchunked(_sc_segment_sum(_chunked(h), sidx, didx, zrows))
        h, f = _layer(i, h, agg, hres, p)
        if f is not None:
            feats.append(f)
    xf, a, amax = _attn(feats[0], feats[1], feats[2], p)
    return _pool(xf, a, amax, batch2, p)


# speedup vs baseline: 5.8832x; 1.1357x over previous
import functools

import jax
import jax.numpy as jnp
from jax import lax
from jax.experimental import pallas as pl
from jax.experimental.pallas import tpu as pltpu
from jax.experimental.pallas import tpu_sc as plsc

N = 50000
E = 800000
B = 64
H = 128
OUT = 256
NL = 5
FL = 3
D = (H // 2) * FL

_BN_S = float((1.0 + 1e-5) ** -0.5)
NEG_BIG = float(jnp.finfo(jnp.float32).min)

NSUB = 16
NCORE = 2
CHUNK_W = 32
NCHUNK = H // CHUNK_W
BATCH = 125
KSUB = 2
G = KSUB * BATCH
EP = E // NSUB
NSTEP = EP // G
TSTEP = 20
NIBLK = NSTEP // TSTEP
STRIPE = 3200
STRIPE_LAST = N - STRIPE * (NSUB - 1)


def _agg_body(h4, sidx, didx, zrows, out, acc,
              rows0, rows1, svb, dvb, sem0, sem1):
    cid = lax.axis_index("c")
    sid = lax.axis_index("s")
    last = NSUB - 1
    bufs = ((rows0, sem0), (rows1, sem1))
    for step in range(NCHUNK // NCORE):
        c = cid + NCORE * step
        base_s = c * (E // BATCH) + sid * (EP // BATCH)
        base_d = sid * (EP // BATCH)

        @pl.when(sid < last)
        def _():
            pltpu.sync_copy(zrows, acc.at[pl.ds(sid * STRIPE, STRIPE)])

        @pl.when(sid == last)
        def _():
            pltpu.sync_copy(zrows.at[pl.ds(0, STRIPE_LAST)],
                            acc.at[pl.ds(sid * STRIPE, STRIPE_LAST)])

        plsc.subcore_barrier()

        def fire(t, buf):
            rows, sem = buf
            for j in range(KSUB):
                pltpu.async_copy(h4.at[svb.at[t * KSUB + j]],
                                 rows.at[pl.ds(j * BATCH, BATCH)], sem)

        def drain_scatter(t, buf):
            rows, sem = buf
            for j in range(KSUB):
                pltpu.make_async_copy(h4.at[svb.at[t * KSUB + j]],
                                      rows.at[pl.ds(j * BATCH, BATCH)],
                                      sem).wait()
            for j in range(KSUB):
                pltpu.sync_copy(rows.at[pl.ds(j * BATCH, BATCH)],
                                acc.at[dvb.at[t * KSUB + j]], add=True)

        def block_body(b, carry):
            pltpu.sync_copy(
                sidx.at[pl.ds(base_s + b * TSTEP * KSUB, TSTEP * KSUB)], svb)
            pltpu.sync_copy(
                didx.at[pl.ds(base_d + b * TSTEP * KSUB, TSTEP * KSUB)], dvb)
            fire(0, bufs[0])

            def pair(u, carry2):
                fire(2 * u + 1, bufs[1])
                drain_scatter(2 * u, bufs[0])

                @pl.when(u < TSTEP // 2 - 1)
                def _():
                    fire(2 * u + 2, bufs[0])

                drain_scatter(2 * u + 1, bufs[1])
                return carry2

            lax.fori_loop(0, TSTEP // 2, pair, None)
            return carry

        lax.fori_loop(0, NIBLK, block_body, None)
        plsc.subcore_barrier()

        @pl.when(sid < last)
        def _():
            pltpu.sync_copy(acc.at[pl.ds(sid * STRIPE, STRIPE)],
                            out.at[pl.ds(sid * STRIPE, STRIPE), c])

        @pl.when(sid == last)
        def _():
            pltpu.sync_copy(acc.at[pl.ds(sid * STRIPE, STRIPE_LAST)],
                            out.at[pl.ds(sid * STRIPE, STRIPE_LAST), c])

        plsc.subcore_barrier()


def _sc_segment_sum(h4, sidx, didx, zrows):
    mesh = plsc.VectorSubcoreMesh(core_axis_name="c", subcore_axis_name="s")
    fn = pl.kernel(
        _agg_body,
        out_type=jax.ShapeDtypeStruct((N, NCHUNK, CHUNK_W), jnp.float32),
        mesh=mesh,
        scratch_types=[
            pltpu.VMEM_SHARED((N, CHUNK_W), jnp.float32),
            pltpu.VMEM((G, CHUNK_W), jnp.float32),
            pltpu.VMEM((G, CHUNK_W), jnp.float32),
            pltpu.VMEM((TSTEP * KSUB, BATCH), jnp.int32),
            pltpu.VMEM((TSTEP * KSUB, BATCH), jnp.int32),
            pltpu.SemaphoreType.DMA,
            pltpu.SemaphoreType.DMA,
        ],
        compiler_params=pltpu.CompilerParams(use_tc_tiling_on_sc=False),
    )
    return fn(h4, sidx, didx, zrows)


BN = 2000
NBLK = N // BN


def _pre_body(pos, x, pw1, pb1, pw2, pb2, wx, wpe, pjb, pjg, pjbb, rw, rb,
              h_out, hres_out):
    pe = jax.nn.relu(jnp.dot(pos[...], pw1[...],
                             preferred_element_type=jnp.float32) + pb1[...])
    pe = jax.nn.relu(jnp.dot(pe, pw2[...],
                             preferred_element_type=jnp.float32) + pb2[...])
    hh = (jnp.dot(x[...], wx[...], preferred_element_type=jnp.float32)
          + jnp.dot(pe, wpe[...], preferred_element_type=jnp.float32)
          + pjb[...])
    hh = jax.nn.relu(pjg[...] * hh * _BN_S + pjbb[...])
    h_out[...] = hh
    hres_out[...] = jnp.dot(hh, rw[...], preferred_element_type=jnp.float32) + rb[...]


def _pre(x, positions, p):
    row = lambda i: (i, 0)
    full = lambda i: (0, 0)
    return pl.pallas_call(
        _pre_body,
        grid=(NBLK,),
        in_specs=[
            pl.BlockSpec((BN, 3), row),
            pl.BlockSpec((BN, 6), row),
            pl.BlockSpec((3, 16), full),
            pl.BlockSpec((1, 16), full),
            pl.BlockSpec((16, 16), full),
            pl.BlockSpec((1, 16), full),
            pl.BlockSpec((6, H), full),
            pl.BlockSpec((16, H), full),
            pl.BlockSpec((1, H), full),
            pl.BlockSpec((1, H), full),
            pl.BlockSpec((1, H), full),
            pl.BlockSpec((H, H), full),
            pl.BlockSpec((1, H), full),
        ],
        out_specs=[
            pl.BlockSpec((BN, H), row),
            pl.BlockSpec((BN, H), row),
        ],
        out_shape=[
            jax.ShapeDtypeStruct((N, H), jnp.float32),
            jax.ShapeDtypeStruct((N, H), jnp.float32),
        ],
        compiler_params=pltpu.CompilerParams(dimension_semantics=("arbitrary",)),
    )(positions, x,
      p['pos_W1'], p['pos_b1'][None, :], p['pos_W2'], p['pos_b2'][None, :],
      p['proj_W'][:6], p['proj_W'][6:], p['proj_b'][None, :],
      p['proj_g'][None, :], p['proj_bb'][None, :],
      p['res_W'], p['res_b'][None, :])


def _layer_body(first, fused, h, agg, w1, b1, w2, b2, gg, gbb, bg, bb, *rest):
    if first:
        hres, rest = rest[0], rest[1:]
    if fused:
        fw, fb, h_out, f_out = rest
    else:
        (h_out,) = rest
    t = h[...] + agg[...]
    t = jax.nn.relu(jnp.dot(t, w1[...], preferred_element_type=jnp.float32) + b1[...])
    t = jax.nn.relu(jnp.dot(t, w2[...], preferred_element_type=jnp.float32) + b2[...])
    t = gg[...] * t * _BN_S + gbb[...]
    if first:
        t = t + hres[...]
    t = bg[...] * t * _BN_S + bb[...]
    hn = jax.nn.relu(t)
    h_out[...] = hn
    if fused:
        f_out[...] = jax.nn.relu(
            jnp.dot(hn, fw[...], preferred_element_type=jnp.float32) + fb[...])


def _layer(i, h, agg, hres, p):
    first = i == 0
    fused = i >= NL - FL
    row = lambda g: (g, 0)
    full = lambda g: (0, 0)
    in_specs = [
        pl.BlockSpec((BN, H), row),
        pl.BlockSpec((BN, H), row),
        pl.BlockSpec((H, H), full),
        pl.BlockSpec((1, H), full),
        pl.BlockSpec((H, H), full),
        pl.BlockSpec((1, H), full),
        pl.BlockSpec((1, H), full),
        pl.BlockSpec((1, H), full),
        pl.BlockSpec((1, H), full),
        pl.BlockSpec((1, H), full),
    ]
    args = [h, agg,
            p['gin_W1'][i], p['gin_b1'][i][None, :],
            p['gin_W2'][i], p['gin_b2'][i][None, :],
            p['gin_g'][i][None, :], p['gin_bb'][i][None, :],
            p['bn_g'][i][None, :], p['bn_b'][i][None, :]]
    if first:
        in_specs.append(pl.BlockSpec((BN, H), row))
        args.append(hres)
    out_specs = [pl.BlockSpec((BN, H), row)]
    out_shape = [jax.ShapeDtypeStruct((N, H), jnp.float32)]
    if fused:
        j = i - (NL - FL)
        in_specs.append(pl.BlockSpec((H, H // 2), full))
        in_specs.append(pl.BlockSpec((1, H // 2), full))
        args.append(p['fus_W'][j])
        args.append(p['fus_b'][j][None, :])
        out_specs.append(pl.BlockSpec((BN, H // 2), row))
        out_shape.append(jax.ShapeDtypeStruct((N, H // 2), jnp.float32))
    res = pl.pallas_call(
        functools.partial(_layer_body, first, fused),
        grid=(NBLK,),
        in_specs=in_specs,
        out_specs=out_specs,
        out_shape=out_shape,
        compiler_params=pltpu.CompilerParams(dimension_semantics=("arbitrary",)),
    )(*args)
    return res if fused else (res[0], None)


def _attn_body(f0, f1, f2, w1, b1, w2, b2, xf_out, a_out, amax_out, am):
    i = pl.program_id(0)
    xf = jnp.concatenate([f0[...], f1[...], f2[...]], axis=1)
    xf_out[...] = xf
    a = jnp.tanh(jnp.dot(xf, w1[...], preferred_element_type=jnp.float32) + b1[...])
    a = jnp.dot(a, w2[...], preferred_element_type=jnp.float32) + b2[...]
    a_out[...] = a
    m = jnp.max(a)

    @pl.when(i == 0)
    def _():
        am[0, 0] = m

    @pl.when(i > 0)
    def _():
        am[0, 0] = jnp.maximum(am[0, 0], m)

    @pl.when(i == NBLK - 1)
    def _():
        amax_out[...] = jnp.full((1, 1), am[0, 0], jnp.float32)


def _attn(f0, f1, f2, p):
    row = lambda g: (g, 0)
    full = lambda g: (0, 0)
    return pl.pallas_call(
        _attn_body,
        grid=(NBLK,),
        in_specs=[
            pl.BlockSpec((BN, H // 2), row),
            pl.BlockSpec((BN, H // 2), row),
            pl.BlockSpec((BN, H // 2), row),
            pl.BlockSpec((D, H), full),
            pl.BlockSpec((1, H), full),
            pl.BlockSpec((H, 1), full),
            pl.BlockSpec((1, 1), full),
        ],
        out_specs=[
            pl.BlockSpec((BN, D), row),
            pl.BlockSpec((BN, 1), row),
            pl.BlockSpec((1, 1), full),
        ],
        out_shape=[
            jax.ShapeDtypeStruct((N, D), jnp.float32),
            jax.ShapeDtypeStruct((N, 1), jnp.float32),
            jax.ShapeDtypeStruct((1, 1), jnp.float32),
        ],
        scratch_shapes=[pltpu.SMEM((1, 1), jnp.float32)],
        compiler_params=pltpu.CompilerParams(dimension_semantics=("arbitrary",)),
    )(f0, f1, f2, p['attn_W1'], p['attn_b1'][None, :],
      p['attn_W2'], p['attn_b2'][None, :])


def _pool_body(xf, a, amax, bid, ow1, ob1, ow2, ob2, o_out,
               s_wsum, s_sum, s_max, s_cnt, s_es):
    i = pl.program_id(0)

    @pl.when(i == 0)
    def _():
        s_wsum[...] = jnp.zeros_like(s_wsum)
        s_sum[...] = jnp.zeros_like(s_sum)
        s_max[...] = jnp.full_like(s_max, -jnp.inf)
        s_cnt[...] = jnp.zeros_like(s_cnt)
        s_es[0, 0] = 0.0

    xfv = xf[...]
    ids = bid[...]
    e = jnp.exp(a[...] - amax[0, 0])
    oh = (ids == lax.broadcasted_iota(jnp.int32, (1, B), 1)).astype(jnp.float32)
    ct = (((0,), (0,)), ((), ()))
    s_wsum[...] += lax.dot_general(oh, xfv * e, ct,
                                   preferred_element_type=jnp.float32)
    s_sum[...] += lax.dot_general(oh, xfv, ct,
                                  preferred_element_type=jnp.float32)
    s_cnt[...] += jnp.broadcast_to(jnp.sum(oh, axis=0)[:, None], s_cnt.shape)
    s_es[0, 0] += jnp.sum(e)
    bmin = jnp.min(ids)
    bmax = jnp.max(ids)
    for b in range(B):
        @pl.when((bmin <= b) & (b <= bmax))
        def _():
            cur = jnp.max(jnp.where(ids == b, xfv, -jnp.inf), axis=0)
            s_max[pl.ds(b, 1), :] = jnp.maximum(s_max[pl.ds(b, 1), :], cur[None, :])

    @pl.when(i == NBLK - 1)
    def _():
        cnt = jnp.maximum(s_cnt[...][:, :1], 1.0)
        wsum = s_wsum[...] / (s_es[0, 0] * cnt)
        gmean = s_sum[...] / cnt
        pooled = jnp.concatenate([wsum, gmean, s_max[...]], axis=1)
        o = jax.nn.relu(jnp.dot(pooled, ow1[...],
                                preferred_element_type=jnp.float32) + ob1[...])
        o_out[...] = jax.nn.relu(jnp.dot(o, ow2[...],
                                         preferred_element_type=jnp.float32) + ob2[...])


def _pool(xf, a, amax, batch2, p):
    row = lambda g: (g, 0)
    full = lambda g: (0, 0)
    return pl.pallas_call(
        _pool_body,
        grid=(NBLK,),
        in_specs=[
            pl.BlockSpec((BN, D), row),
            pl.BlockSpec((BN, 1), row),
            pl.BlockSpec(memory_space=pltpu.SMEM),
            pl.BlockSpec((BN, 1), row),
            pl.BlockSpec((3 * D, 2 * H), full),
            pl.BlockSpec((1, 2 * H), full),
            pl.BlockSpec((2 * H, OUT), full),
            pl.BlockSpec((1, OUT), full),
        ],
        out_specs=pl.BlockSpec((B, OUT), full),
        out_shape=jax.ShapeDtypeStruct((B, OUT), jnp.float32),
        scratch_shapes=[
            pltpu.VMEM((B, D), jnp.float32),
            pltpu.VMEM((B, D), jnp.float32),
            pltpu.VMEM((B, D), jnp.float32),
            pltpu.VMEM((B, 128), jnp.float32),
            pltpu.SMEM((1, 1), jnp.float32),
        ],
        compiler_params=pltpu.CompilerParams(dimension_semantics=("arbitrary",)),
    )(xf, a, amax, batch2,
      p['out_W1'], p['out_b1'][None, :], p['out_W2'], p['out_b2'][None, :])


def kernel(x, edge_index, positions, batch, params):
    p = params
    src = edge_index[0].astype(jnp.int32)
    dst = edge_index[1].astype(jnp.int32)
    offs = jnp.arange(NCHUNK, dtype=jnp.int32)[:, None]
    sidx = (src[None, :] * NCHUNK + offs).reshape(NCHUNK * E // BATCH, BATCH)
    didx = dst.reshape(E // BATCH, BATCH)
    zrows = jnp.zeros((STRIPE, CHUNK_W), jnp.float32)
    batch2 = batch.astype(jnp.int32).reshape(N, 1)

    h, hres = _pre(x, positions, p)
    feats = []
    for i in range(NL):
        agg = _sc_segment_sum(h.reshape(NCHUNK * N, CHUNK_W),
                              sidx, didx, zrows).reshape(N, H)
        h, f = _layer(i, h, agg, hres, p)
        if f is not None:
            feats.append(f)
    xf, a, amax = _attn(feats[0], feats[1], feats[2], p)
    return _pool(xf, a, amax, batch2, p)
